# Initial kernel scaffold; baseline (speedup 1.0000x reference)
import jax
import jax.numpy as jnp
from jax.experimental import pallas as pl


def kernel(x, table):
    raise NotImplementedError("write your pallas kernel here")


# speedup vs baseline: 1.8467x; 1.8467x over previous
import functools

import jax
import jax.numpy as jnp
from jax import lax
from jax.experimental import pallas as pl
from jax.experimental.pallas import tpu as pltpu
from jax.experimental.pallas import tpu_sc as plsc

DIM = 64
LANES = 128
KROWS = 8
CHUNK = KROWS * LANES


def _emb_body(idx_hbm, table_hbm, out_hbm, idx_v, rows_v, sem):
    c = lax.axis_index("c")
    s = lax.axis_index("s")
    wid = s * 2 + c
    n_rows128 = idx_hbm.shape[0]
    rows_per_w = n_rows128 // 32
    n_chunks = rows_per_w // KROWS
    base_row = wid * rows_per_w

    def chunk(ci, carry):
        r0 = base_row + ci * KROWS
        pltpu.sync_copy(idx_hbm.at[pl.ds(r0, KROWS)], idx_v)
        cps = [
            pltpu.async_copy(
                table_hbm.at[idx_v.at[j]],
                rows_v.at[pl.ds(j * LANES, LANES)],
                sem,
            )
            for j in range(KROWS)
        ]
        for cp in cps:
            cp.wait()
        pltpu.sync_copy(rows_v, out_hbm.at[pl.ds(r0 * LANES, CHUNK)])
        return carry

    lax.fori_loop(0, n_chunks, chunk, 0)


def kernel(x, table):
    B, L = x.shape
    N = B * L
    idx = x.reshape(N // LANES, LANES).astype(jnp.int32)

    mesh = plsc.VectorSubcoreMesh(core_axis_name="c", subcore_axis_name="s")
    emb = functools.partial(
        pl.kernel,
        mesh=mesh,
        out_type=jax.ShapeDtypeStruct((N, DIM), jnp.float32),
        scratch_types=[
            pltpu.VMEM((KROWS, LANES), jnp.int32),
            pltpu.VMEM((CHUNK, DIM), jnp.float32),
            pltpu.SemaphoreType.DMA,
        ],
        compiler_params=pltpu.CompilerParams(use_tc_tiling_on_sc=False),
    )(_emb_body)

    out = emb(idx, table)
    return out.reshape(B, L, DIM)


# speedup vs baseline: 1.8745x; 1.0151x over previous
import functools

import jax
import jax.numpy as jnp
from jax import lax
from jax.experimental import pallas as pl
from jax.experimental.pallas import tpu as pltpu
from jax.experimental.pallas import tpu_sc as plsc

DIM = 64
LANES = 128
KROWS = 5
CHUNK = KROWS * LANES


def _emb_body(idx_hbm, table_hbm, out_hbm,
              idx_v, rows_v, sem_i0, sem_i1, sem_o0, sem_o1, sem_g):
    sem_i = (sem_i0, sem_i1)
    sem_o = (sem_o0, sem_o1)
    wid = lax.axis_index("s") * 2 + lax.axis_index("c")
    n_rows128 = idx_hbm.shape[0]
    rows_per_w = n_rows128 // 32
    n_chunks = rows_per_w // KROWS
    base_row = wid * rows_per_w

    def idx_copy(g, b):
        return pltpu.async_copy(
            idx_hbm.at[pl.ds(base_row + g * KROWS, KROWS)],
            idx_v.at[b], sem_i[b])

    def out_copy(g, b):
        return pltpu.async_copy(
            rows_v.at[b],
            out_hbm.at[pl.ds((base_row + g * KROWS) * LANES, CHUNK)],
            sem_o[b])

    def wait_idx(b):
        pltpu.make_async_copy(
            idx_hbm.at[pl.ds(base_row, KROWS)], idx_v.at[b], sem_i[b]).wait()

    def wait_out(b):
        pltpu.make_async_copy(
            rows_v.at[b], out_hbm.at[pl.ds(base_row * LANES, CHUNK)],
            sem_o[b]).wait()

    def gather(b):
        cps = [
            pltpu.async_copy(
                table_hbm.at[idx_v.at[b, j]],
                rows_v.at[b, pl.ds(j * LANES, LANES)],
                sem_g)
            for j in range(KROWS)
        ]
        for cp in cps:
            cp.wait()

    idx_copy(0, 0)
    idx_copy(1, 1)
    for b in range(2):
        wait_idx(b)
        gather(b)
        out_copy(b, b)
        idx_copy(b + 2, b)

    def step(t, carry):
        for b in range(2):
            g = 2 * t + b
            wait_out(b)
            wait_idx(b)
            gather(b)
            out_copy(g, b)

            @pl.when(g + 2 < n_chunks)
            def _():
                idx_copy(g + 2, b)
        return carry

    lax.fori_loop(1, n_chunks // 2, step, 0)
    for b in range(2):
        wait_out(b)


def kernel(x, table):
    B, L = x.shape
    N = B * L
    idx = x.reshape(N // LANES, LANES).astype(jnp.int32)

    mesh = plsc.VectorSubcoreMesh(core_axis_name="c", subcore_axis_name="s")
    emb = functools.partial(
        pl.kernel,
        mesh=mesh,
        out_type=jax.ShapeDtypeStruct((N, DIM), jnp.float32),
        scratch_types=[
            pltpu.VMEM((2, KROWS, LANES), jnp.int32),
            pltpu.VMEM((2, CHUNK, DIM), jnp.float32),
            pltpu.SemaphoreType.DMA,
            pltpu.SemaphoreType.DMA,
            pltpu.SemaphoreType.DMA,
            pltpu.SemaphoreType.DMA,
            pltpu.SemaphoreType.DMA,
        ],
        compiler_params=pltpu.CompilerParams(use_tc_tiling_on_sc=False),
    )(_emb_body)

    out = emb(idx, table)
    return out.reshape(B, L, DIM)
